# baseline (speedup 1.0000x reference)
import jax
import jax.numpy as jnp
from jax import lax
from jax.experimental import pallas as pl
from jax.experimental.pallas import tpu as pltpu
from jax.experimental.pallas import tpu_sc as plsc

NUM_EMB = 100000
EMB_DIM = 64
BATCH = 16384

_NC = 2
_NS = 16
_NW = _NC * _NS

_TOTAL = BATCH * 2
_PER_W = _TOTAL // _NW
_CHUNK = 128
_NCHUNK = _PER_W // _CHUNK


def _gather_kernel(idx_hbm, table_hbm, out_hbm, idx_v, rows_v, sem):
    wid = lax.axis_index("s") * _NC + lax.axis_index("c")
    pltpu.sync_copy(idx_hbm.at[pl.ds(wid * _NCHUNK, _NCHUNK)], idx_v)
    for j in range(_NCHUNK):
        pltpu.async_copy(
            table_hbm.at[idx_v.at[j]],
            rows_v.at[pl.ds(j * _CHUNK, _CHUNK)],
            sem,
        )
    pltpu.make_async_copy(
        table_hbm.at[idx_v.at[0]], rows_v.at[pl.ds(0, _CHUNK)], sem
    ).wait()
    for j in range(1, _NCHUNK):
        pltpu.make_async_copy(
            table_hbm.at[idx_v.at[j]],
            rows_v.at[pl.ds(j * _CHUNK, _CHUNK)],
            sem,
        ).wait()
    pltpu.sync_copy(rows_v, out_hbm.at[pl.ds(wid * _PER_W, _PER_W)])


@jax.jit
def kernel(labels, att_baseline):
    idx = labels.reshape(_TOTAL // _CHUNK, _CHUNK).astype(jnp.int32)
    mesh = plsc.VectorSubcoreMesh(core_axis_name="c", subcore_axis_name="s")
    out = pl.kernel(
        _gather_kernel,
        out_type=jax.ShapeDtypeStruct((_TOTAL, EMB_DIM), jnp.float32),
        mesh=mesh,
        scratch_types=[
            pltpu.VMEM((_NCHUNK, _CHUNK), jnp.int32),
            pltpu.VMEM((_PER_W, EMB_DIM), jnp.float32),
            pltpu.SemaphoreType.DMA,
        ],
        compiler_params=pltpu.CompilerParams(use_tc_tiling_on_sc=False),
    )(idx, att_baseline)
    return out.reshape(BATCH, 2, EMB_DIM)


# speedup vs baseline: 1.3738x; 1.3738x over previous
import jax
import jax.numpy as jnp
from jax import lax
from jax.experimental import pallas as pl
from jax.experimental.pallas import tpu as pltpu
from jax.experimental.pallas import tpu_sc as plsc

NUM_EMB = 100000
EMB_DIM = 64
BATCH = 16384

_NW = 32
_NT = 128
_NPH = 8
_TQ = _NT // _NPH


def _body(idx_hbm, tab_hbm, out_hbm, row_v, idx_v, out_v):
    wid = lax.axis_index("s") * 2 + lax.axis_index("c")

    def do_feature(c):
        pltpu.sync_copy(tab_hbm.at[c], row_v)
        cr = c // 8
        ci = c % 8

        def phase(ph, carry):
            pltpu.sync_copy(idx_hbm.at[pl.ds(2 * _TQ * ph, 2 * _TQ)], idx_v)

            for tq in range(_TQ):
                for p in range(2):
                    for g in range(8):
                        iv = idx_v[2 * tq + p, pl.ds(16 * g, 16)]
                        vals = plsc.load_gather(row_v, [iv])
                        out_v[p, tq, pl.ds(16 * g, 16)] = vals
            for p in range(2):
                pltpu.sync_copy(
                    out_v.at[p], out_hbm.at[p, cr, pl.ds(_TQ * ph, _TQ), ci, :]
                )
            return carry

        lax.fori_loop(0, _NPH, phase, 0)

    do_feature(wid)
    do_feature(wid + 32)


@jax.jit
def kernel(labels, att_baseline):
    tab_t = att_baseline.T
    idx = labels.reshape(128, 128, 2).transpose(0, 2, 1).reshape(256, 128)
    mesh = plsc.VectorSubcoreMesh(core_axis_name="c", subcore_axis_name="s")
    a5 = pl.kernel(
        _body,
        out_type=jax.ShapeDtypeStruct((2, 8, _NT, 8, 128), jnp.float32),
        mesh=mesh,
        scratch_types=[
            pltpu.VMEM((NUM_EMB,), jnp.float32),
            pltpu.VMEM((2 * _TQ, 128), jnp.int32),
            pltpu.VMEM((2, _TQ, 128), jnp.float32),
        ],
        compiler_params=pltpu.CompilerParams(
            use_tc_tiling_on_sc=False, needs_layout_passes=False
        ),
    )(idx, tab_t)
    return a5.transpose(2, 4, 0, 1, 3).reshape(BATCH, 2, EMB_DIM)


# speedup vs baseline: 1.6668x; 1.2132x over previous
import jax
import jax.numpy as jnp
from jax import lax
from jax.experimental import pallas as pl
from jax.experimental.pallas import tpu as pltpu
from jax.experimental.pallas import tpu_sc as plsc

NUM_EMB = 100000
EMB_DIM = 64
BATCH = 16384

_NPH = 8


def _body(idx_hbm, tab_hbm, out_hbm, row_v, idx_v0, idx_v1, out_v0, out_v1,
          idx_sem, out_sem0, out_sem1):
    wid = lax.axis_index("s") * 2 + lax.axis_index("c")
    idx_bufs = (idx_v0, idx_v1)
    out_bufs = (out_v0, out_v1)
    out_sems = (out_sem0, out_sem1)
    pending = {0: [], 1: []}

    def idx_src(ph):
        return idx_hbm.at[pl.ds(32 * ph, 32)]

    for c in (wid, wid + 32):
        pltpu.sync_copy(tab_hbm.at[c], row_v)
        cr = c // 8
        ci = c % 8
        for ph in range(_NPH):
            b = ph & 1
            if ph == 0:
                pltpu.sync_copy(idx_src(0), idx_bufs[0])
            else:
                pltpu.make_async_copy(idx_src(ph), idx_bufs[b], idx_sem).wait()
            if ph < _NPH - 1:
                pltpu.async_copy(idx_src(ph + 1), idx_bufs[1 - b], idx_sem)
            for src, dst in pending[b]:
                pltpu.make_async_copy(src, dst, out_sems[b]).wait()
            pending[b] = []

            idx_buf = idx_bufs[b]
            out_buf = out_bufs[b]

            @plsc.parallel_loop(0, 256, unroll=8)
            def _(i):
                r = i >> 3
                tq = i >> 4
                p = (i >> 3) & 1
                g = (i & 7) << 4
                iv = idx_buf[r, pl.ds(g, 16)]
                out_buf[p, tq, pl.ds(g, 16)] = plsc.load_gather(row_v, [iv])

            for p in range(2):
                src = out_buf.at[p]
                dst = out_hbm.at[p, cr, pl.ds(16 * ph, 16), ci, :]
                pltpu.async_copy(src, dst, out_sems[b])
                pending[b].append((src, dst))

    for b in range(2):
        for src, dst in pending[b]:
            pltpu.make_async_copy(src, dst, out_sems[b]).wait()


@jax.jit
def kernel(labels, att_baseline):
    tab_t = att_baseline.T
    idx = labels.reshape(128, 128, 2).transpose(0, 2, 1).reshape(256, 128)
    mesh = plsc.VectorSubcoreMesh(core_axis_name="c", subcore_axis_name="s")
    a5 = pl.kernel(
        _body,
        out_type=jax.ShapeDtypeStruct((2, 8, 128, 8, 128), jnp.float32),
        mesh=mesh,
        scratch_types=[
            pltpu.VMEM((NUM_EMB,), jnp.float32),
            pltpu.VMEM((32, 128), jnp.int32),
            pltpu.VMEM((32, 128), jnp.int32),
            pltpu.VMEM((2, 16, 128), jnp.float32),
            pltpu.VMEM((2, 16, 128), jnp.float32),
            pltpu.SemaphoreType.DMA,
            pltpu.SemaphoreType.DMA,
            pltpu.SemaphoreType.DMA,
        ],
        compiler_params=pltpu.CompilerParams(
            use_tc_tiling_on_sc=False, needs_layout_passes=False
        ),
    )(idx, tab_t)
    return a5.transpose(2, 4, 0, 1, 3).reshape(BATCH, 2, EMB_DIM)


# speedup vs baseline: 1.7988x; 1.0792x over previous
import jax
import jax.numpy as jnp
from jax import lax
from jax.experimental import pallas as pl
from jax.experimental.pallas import tpu as pltpu
from jax.experimental.pallas import tpu_sc as plsc

NUM_EMB = 100000
EMB_DIM = 64
BATCH = 16384

_NPH = 8

_VOC = 99968
_TAIL = NUM_EMB - _VOC
_CH = 6272


def _detile_body(tab_hbm, lin_hbm, stage0, stage1, sem0, sem1):
    w = lax.axis_index("s") * 2 + lax.axis_index("c")
    band = w >> 2
    q = w & 3
    stages = (stage0, stage1)
    sems = (sem0, sem1)

    def run_branch(m0, lengths):
        pending = [None, None]
        off = 0
        for j, length in enumerate(lengths):
            b = j & 1
            m = m0 + off
            off += length
            if pending[b] is not None:
                for ps, pd in pending[b]:
                    pltpu.make_async_copy(ps, pd, sems[b]).wait()
            pltpu.sync_copy(
                tab_hbm.at[pl.ds(band * 8, 8), pl.ds(m, length)],
                stages[b].at[:, pl.ds(0, length)],
            )
            outs = []
            for i in range(8):
                s = stages[b].at[i, pl.ds(0, length)]
                d = lin_hbm.at[pl.ds((band * 8 + i) * _VOC + m, length)]
                pltpu.async_copy(s, d, sems[b])
                outs.append((s, d))
            pending[b] = outs
        for b in range(2):
            if pending[b] is not None:
                for ps, pd in pending[b]:
                    pltpu.make_async_copy(ps, pd, sems[b]).wait()

    @pl.when(q == 0)
    def _():
        run_branch(0, [_CH, _CH, _CH, _CH])

    @pl.when(q > 0)
    def _():
        run_branch(q * 24960 + 128, [_CH, _CH, _CH, 24960 - 3 * _CH])


def _body(idx_hbm, tab_hbm, tail_hbm, out_hbm, row_v, idx_v0, idx_v1,
          out_v0, out_v1, idx_sem, out_sem0, out_sem1):
    wid = lax.axis_index("s") * 2 + lax.axis_index("c")
    idx_bufs = (idx_v0, idx_v1)
    out_bufs = (out_v0, out_v1)
    out_sems = (out_sem0, out_sem1)
    pending = {0: [], 1: []}

    def idx_src(ph):
        return idx_hbm.at[pl.ds(32 * ph, 32)]

    for c in (wid, wid + 32):
        pltpu.sync_copy(tab_hbm.at[c], row_v.at[pl.ds(0, _VOC)])
        pltpu.sync_copy(
            tail_hbm.at[pl.ds(c * _TAIL, _TAIL)], row_v.at[pl.ds(_VOC, _TAIL)]
        )
        cr = c // 8
        ci = c % 8
        for ph in range(_NPH):
            b = ph & 1
            if ph == 0:
                pltpu.sync_copy(idx_src(0), idx_bufs[0])
            else:
                pltpu.make_async_copy(idx_src(ph), idx_bufs[b], idx_sem).wait()
            if ph < _NPH - 1:
                pltpu.async_copy(idx_src(ph + 1), idx_bufs[1 - b], idx_sem)
            for src, dst in pending[b]:
                pltpu.make_async_copy(src, dst, out_sems[b]).wait()
            pending[b] = []

            idx_buf = idx_bufs[b]
            out_buf = out_bufs[b]

            @plsc.parallel_loop(0, 256, unroll=8)
            def _(i):
                r = i >> 3
                tq = i >> 4
                p = (i >> 3) & 1
                g = (i & 7) << 4
                iv = idx_buf[r, pl.ds(g, 16)]
                out_buf[p, tq, pl.ds(g, 16)] = plsc.load_gather(row_v, [iv])

            for p in range(2):
                src = out_buf.at[p]
                dst = out_hbm.at[p, cr, pl.ds(16 * ph, 16), ci, :]
                pltpu.async_copy(src, dst, out_sems[b])
                pending[b].append((src, dst))

    for b in range(2):
        for src, dst in pending[b]:
            pltpu.make_async_copy(src, dst, out_sems[b]).wait()


@jax.jit
def kernel(labels, att_baseline):
    tab_t = att_baseline.T
    idx = labels.reshape(128, 128, 2).transpose(0, 2, 1).reshape(256, 128)
    mesh = plsc.VectorSubcoreMesh(core_axis_name="c", subcore_axis_name="s")
    lin = pl.kernel(
        _detile_body,
        out_type=jax.ShapeDtypeStruct((64 * _VOC,), jnp.float32),
        mesh=mesh,
        scratch_types=[
            pltpu.VMEM((8, _CH), jnp.float32),
            pltpu.VMEM((8, _CH), jnp.float32),
            pltpu.SemaphoreType.DMA,
            pltpu.SemaphoreType.DMA,
        ],
    )(tab_t)
    tab_lin = lin.reshape(64, _VOC)
    tail = tab_t[:, _VOC:].reshape(64 * _TAIL)
    a5 = pl.kernel(
        _body,
        out_type=jax.ShapeDtypeStruct((2, 8, 128, 8, 128), jnp.float32),
        mesh=mesh,
        scratch_types=[
            pltpu.VMEM((NUM_EMB,), jnp.float32),
            pltpu.VMEM((32, 128), jnp.int32),
            pltpu.VMEM((32, 128), jnp.int32),
            pltpu.VMEM((2, 16, 128), jnp.float32),
            pltpu.VMEM((2, 16, 128), jnp.float32),
            pltpu.SemaphoreType.DMA,
            pltpu.SemaphoreType.DMA,
            pltpu.SemaphoreType.DMA,
        ],
        compiler_params=pltpu.CompilerParams(
            use_tc_tiling_on_sc=False, needs_layout_passes=False
        ),
    )(idx, tab_lin, tail)
    return a5.transpose(2, 4, 0, 1, 3).reshape(BATCH, 2, EMB_DIM)


# speedup vs baseline: 1.8668x; 1.0378x over previous
import jax
import jax.numpy as jnp
from jax import lax
from jax.experimental import pallas as pl
from jax.experimental.pallas import tpu as pltpu
from jax.experimental.pallas import tpu_sc as plsc

NUM_EMB = 100000
EMB_DIM = 64
BATCH = 16384

_NPH = 8

_VOC = 99968
_TAIL = NUM_EMB - _VOC
_CH = 6272


def _detile_body(tab_hbm, lin_hbm, stage0, stage1, sem0, sem1, rsem0, rsem1):
    w = lax.axis_index("s") * 2 + lax.axis_index("c")
    band = w >> 2
    q = w & 3
    stages = (stage0, stage1)
    sems = (sem0, sem1)

    def run_branch(m0, lengths):
        n = len(lengths)
        offs = []
        off = 0
        for length in lengths:
            offs.append(off)
            off += length
        rsems = (rsem0, rsem1)

        def read_pair(j):
            return (
                tab_hbm.at[pl.ds(band * 8, 8), pl.ds(m0 + offs[j], lengths[j])],
                stages[j & 1].at[:, pl.ds(0, lengths[j])],
            )

        def writes(j):
            m = m0 + offs[j]
            b = j & 1
            return [
                (
                    stages[b].at[i, pl.ds(0, lengths[j])],
                    lin_hbm.at[pl.ds((band * 8 + i) * _VOC + m, lengths[j])],
                )
                for i in range(8)
            ]

        for j in (0, 1):
            s, d = read_pair(j)
            pltpu.async_copy(s, d, rsems[j & 1])
        tail_pending = []
        for j in range(n):
            b = j & 1
            s, d = read_pair(j)
            pltpu.make_async_copy(s, d, rsems[b]).wait()
            ws = writes(j)
            for s, d in ws:
                pltpu.async_copy(s, d, sems[b])
            if j + 2 < n:
                for s, d in ws:
                    pltpu.make_async_copy(s, d, sems[b]).wait()
                s, d = read_pair(j + 2)
                pltpu.async_copy(s, d, rsems[b])
            else:
                tail_pending.append((ws, b))
        for ws, b in tail_pending:
            for s, d in ws:
                pltpu.make_async_copy(s, d, sems[b]).wait()

    @pl.when(q == 0)
    def _():
        run_branch(0, [_CH, _CH, _CH, _CH])

    @pl.when(q > 0)
    def _():
        run_branch(q * 24960 + 128, [_CH, _CH, _CH, 24960 - 3 * _CH])


def _body(idx_hbm, tab_hbm, tail_hbm, out_hbm, row_v, idx_v0, idx_v1,
          out_v0, out_v1, idx_sem, out_sem0, out_sem1):
    wid = lax.axis_index("s") * 2 + lax.axis_index("c")
    idx_bufs = (idx_v0, idx_v1)
    out_bufs = (out_v0, out_v1)
    out_sems = (out_sem0, out_sem1)
    pending = {0: [], 1: []}

    def idx_src(ph):
        return idx_hbm.at[pl.ds(32 * ph, 32)]

    for c in (wid, wid + 32):
        row_parts = [
            (
                tab_hbm.at[c, pl.ds(k * (_VOC // 4), _VOC // 4)],
                row_v.at[pl.ds(k * (_VOC // 4), _VOC // 4)],
            )
            for k in range(4)
        ]
        row_parts.append(
            (tail_hbm.at[pl.ds(c * _TAIL, _TAIL)], row_v.at[pl.ds(_VOC, _TAIL)])
        )
        for s, d in row_parts:
            pltpu.async_copy(s, d, idx_sem)
        for s, d in row_parts:
            pltpu.make_async_copy(s, d, idx_sem).wait()
        cr = c // 8
        ci = c % 8
        for ph in range(_NPH):
            b = ph & 1
            if ph == 0:
                pltpu.sync_copy(idx_src(0), idx_bufs[0])
            else:
                pltpu.make_async_copy(idx_src(ph), idx_bufs[b], idx_sem).wait()
            if ph < _NPH - 1:
                pltpu.async_copy(idx_src(ph + 1), idx_bufs[1 - b], idx_sem)
            for src, dst in pending[b]:
                pltpu.make_async_copy(src, dst, out_sems[b]).wait()
            pending[b] = []

            idx_buf = idx_bufs[b]
            out_buf = out_bufs[b]

            @plsc.parallel_loop(0, 256, unroll=8)
            def _(i):
                r = i >> 3
                tq = i >> 4
                p = (i >> 3) & 1
                g = (i & 7) << 4
                iv = idx_buf[r, pl.ds(g, 16)]
                out_buf[p, tq, pl.ds(g, 16)] = plsc.load_gather(row_v, [iv])

            for p in range(2):
                src = out_buf.at[p]
                dst = out_hbm.at[p, cr, pl.ds(16 * ph, 16), ci, :]
                pltpu.async_copy(src, dst, out_sems[b])
                pending[b].append((src, dst))

    for b in range(2):
        for src, dst in pending[b]:
            pltpu.make_async_copy(src, dst, out_sems[b]).wait()


@jax.jit
def kernel(labels, att_baseline):
    tab_t = att_baseline.T
    idx = labels.reshape(128, 128, 2).transpose(0, 2, 1).reshape(256, 128)
    mesh = plsc.VectorSubcoreMesh(core_axis_name="c", subcore_axis_name="s")
    lin = pl.kernel(
        _detile_body,
        out_type=jax.ShapeDtypeStruct((64 * _VOC,), jnp.float32),
        mesh=mesh,
        scratch_types=[
            pltpu.VMEM((8, _CH), jnp.float32),
            pltpu.VMEM((8, _CH), jnp.float32),
            pltpu.SemaphoreType.DMA,
            pltpu.SemaphoreType.DMA,
            pltpu.SemaphoreType.DMA,
            pltpu.SemaphoreType.DMA,
        ],
    )(tab_t)
    tab_lin = lin.reshape(64, _VOC)
    tail = tab_t[:, _VOC:].reshape(64 * _TAIL)
    a5 = pl.kernel(
        _body,
        out_type=jax.ShapeDtypeStruct((2, 8, 128, 8, 128), jnp.float32),
        mesh=mesh,
        scratch_types=[
            pltpu.VMEM((NUM_EMB,), jnp.float32),
            pltpu.VMEM((32, 128), jnp.int32),
            pltpu.VMEM((32, 128), jnp.int32),
            pltpu.VMEM((2, 16, 128), jnp.float32),
            pltpu.VMEM((2, 16, 128), jnp.float32),
            pltpu.SemaphoreType.DMA,
            pltpu.SemaphoreType.DMA,
            pltpu.SemaphoreType.DMA,
        ],
        compiler_params=pltpu.CompilerParams(
            use_tc_tiling_on_sc=False, needs_layout_passes=False
        ),
    )(idx, tab_lin, tail)
    return a5.transpose(2, 4, 0, 1, 3).reshape(BATCH, 2, EMB_DIM)
